# baseline (speedup 1.0000x reference)
import functools

import jax
import jax.numpy as jnp
from jax import lax
from jax.experimental import pallas as pl
from jax.experimental.pallas import tpu as pltpu
from jax.experimental.pallas import tpu_sc as plsc

N = 4096
K = 16
DH = 64
EPS = 1e-5
NK = N * K
BIG = 1e30
BIG2 = 2e30

R_KNN = 512
NB_KNN = N // R_KNN


CW = 256
_WIDTHS = (1536, 2560, 4096)


def _select_topk(W, d_ref, idx_ref, lo):
    cols = lax.broadcasted_iota(jnp.int32, (R_KNN, W), 1)
    lanes = lax.broadcasted_iota(jnp.int32, (R_KNN, K), 1)

    def step(k, carry):
        lastv, lasti, idxacc = carry
        dd = d_ref[:, :W]
        sel = (dd > lastv) | ((dd == lastv) & (cols > lasti))
        dm = jnp.where(sel, dd, BIG2)
        m = jnp.min(dm, axis=1, keepdims=True)
        a = jnp.min(jnp.where(dm == m, cols, jnp.int32(N)),
                    axis=1, keepdims=True)
        return m, a, jnp.where(lanes == k, a, idxacc)

    _, _, idxacc = lax.fori_loop(
        0, K, step,
        (jnp.full((R_KNN, 1), -BIG2, jnp.float32),
         jnp.full((R_KNN, 1), -1, jnp.int32),
         jnp.zeros((R_KNN, K), jnp.int32)))
    idx_ref[...] = jnp.minimum(idxacc + lo, N - 1)


def _knn_uv_body(C, x_ref, xt_ref, br_ref, bc_ref, desc_ref, w1_ref, b1_ref,
                 idx_ref, u_ref, v_ref, d_ref):
    i = pl.program_id(0)
    lo = desc_ref[i, 0]
    nch = desc_ref[i, 1]
    xb = x_ref[...]
    br = br_ref[...]
    sq_r = jnp.sum(xb * xb, axis=1, keepdims=True)
    wsel = jnp.where(nch <= _WIDTHS[0] // CW, _WIDTHS[0],
                     jnp.where(nch <= _WIDTHS[1] // CW, _WIDTHS[1],
                               _WIDTHS[2]))

    for c in range(N // CW):
        @pl.when(c < nch)
        def _(c=c):
            off = pl.multiple_of(lo + c * CW, CW)
            xtc = xt_ref[:, pl.ds(off, CW)]
            p = lax.dot_general(xb, xtc, (((1,), (0,)), ((), ())),
                                preferred_element_type=jnp.float32)
            sq_c = jnp.sum(xtc * xtc, axis=0, keepdims=True)
            d = (sq_r + sq_c) - 2.0 * p
            mask = br != bc_ref[:, pl.ds(off, CW)]
            d_ref[:, c * CW:(c + 1) * CW] = jnp.where(mask, BIG, d)

        @pl.when((c >= nch) & (c * CW < wsel))
        def _(c=c):
            d_ref[:, c * CW:(c + 1) * CW] = jnp.full((R_KNN, CW), BIG,
                                                     jnp.float32)

    @pl.when(nch <= _WIDTHS[0] // CW)
    def _():
        _select_topk(_WIDTHS[0], d_ref, idx_ref, lo)

    @pl.when((nch > _WIDTHS[0] // CW) & (nch <= _WIDTHS[1] // CW))
    def _():
        _select_topk(_WIDTHS[1], d_ref, idx_ref, lo)

    @pl.when(nch > _WIDTHS[1] // CW)
    def _():
        _select_topk(_WIDTHS[2], d_ref, idx_ref, lo)

    w1 = w1_ref[...]
    wa = w1[0:C, :] - w1[C:2 * C, :]
    wb = w1[C:2 * C, :]
    u_ref[...] = lax.dot_general(xb, wa, (((1,), (0,)), ((), ())),
                                 preferred_element_type=jnp.float32) + b1_ref[...]
    v_ref[...] = lax.dot_general(xb, wb, (((1,), (0,)), ((), ())),
                                 preferred_element_type=jnp.float32)


def _knn_uv(x, xt, batch_r, batch_c, desc, w1, b1, C):
    return pl.pallas_call(
        functools.partial(_knn_uv_body, C),
        grid=(NB_KNN,),
        in_specs=[
            pl.BlockSpec((R_KNN, C), lambda i: (i, 0)),
            pl.BlockSpec((C, N), lambda i: (0, 0)),
            pl.BlockSpec((R_KNN, 1), lambda i: (i, 0)),
            pl.BlockSpec((1, N), lambda i: (0, 0)),
            pl.BlockSpec(memory_space=pltpu.SMEM),
            pl.BlockSpec((2 * C, DH), lambda i: (0, 0)),
            pl.BlockSpec((1, DH), lambda i: (0, 0)),
        ],
        out_specs=[
            pl.BlockSpec((R_KNN, K), lambda i: (i, 0)),
            pl.BlockSpec((R_KNN, DH), lambda i: (i, 0)),
            pl.BlockSpec((R_KNN, DH), lambda i: (i, 0)),
        ],
        out_shape=[
            jax.ShapeDtypeStruct((N, K), jnp.int32),
            jax.ShapeDtypeStruct((N, DH), jnp.float32),
            jax.ShapeDtypeStruct((N, DH), jnp.float32),
        ],
        scratch_shapes=[pltpu.VMEM((R_KNN, N), jnp.float32)],
    )(x, xt, batch_r, batch_c, desc, w1, b1)


SC_NC = 2
SC_NS = 16
SC_NW = SC_NC * SC_NS
GCH = 128
ROWS_PER_W = NK // SC_NW
N_CHUNKS = ROWS_PER_W // GCH


def _gather_body(v_hbm, idx_hbm, out_hbm, idx_v0, idx_v1, rows_v0, rows_v1,
                 gsem0, gsem1, ssem0, ssem1):
    wid = lax.axis_index("s") * SC_NC + lax.axis_index("c")
    idx_v = (idx_v0, idx_v1)
    rows_v = (rows_v0, rows_v1)
    gsem = (gsem0, gsem1)
    ssem = (ssem0, ssem1)
    base0 = wid * ROWS_PER_W
    pltpu.sync_copy(idx_hbm.at[pl.ds(base0, GCH)], idx_v[0])
    stores = [None, None]
    for c in range(N_CHUNKS):
        b = c & 1
        if stores[b] is not None:
            stores[b].wait()
        g = pltpu.async_copy(v_hbm.at[idx_v[b]], rows_v[b], gsem[b])
        if c + 1 < N_CHUNKS:
            pltpu.sync_copy(idx_hbm.at[pl.ds(base0 + (c + 1) * GCH, GCH)],
                            idx_v[1 - b])
        g.wait()
        st = pltpu.async_copy(rows_v[b], out_hbm.at[pl.ds(base0 + c * GCH, GCH)],
                              ssem[b])
        stores[b] = st
    for st in stores:
        st.wait()


@functools.lru_cache(maxsize=1)
def _build_gather():
    return functools.partial(
        pl.kernel,
        out_type=jax.ShapeDtypeStruct((NK, DH), jnp.float32),
        mesh=plsc.VectorSubcoreMesh(core_axis_name="c", subcore_axis_name="s"),
        scratch_types=[
            pltpu.VMEM((GCH,), jnp.int32),
            pltpu.VMEM((GCH,), jnp.int32),
            pltpu.VMEM((GCH, DH), jnp.float32),
            pltpu.VMEM((GCH, DH), jnp.float32),
            pltpu.SemaphoreType.DMA,
            pltpu.SemaphoreType.DMA,
            pltpu.SemaphoreType.DMA,
            pltpu.SemaphoreType.DMA,
        ],
        compiler_params=pltpu.CompilerParams(use_tc_tiling_on_sc=False),
    )(_gather_body)


def _gather_rows(v, idx_flat):
    return _build_gather()(v, idx_flat)


R_BN = 512
NB_BN = N // R_BN


def _bn_body(vg_ref, u_ref, g_ref, be_ref, w2_ref, b2_ref, out_ref, outt_ref,
             acc_ref, hbuf_ref):
    i = pl.program_id(0)

    @pl.when(i == 0)
    def _():
        acc_ref[...] = jnp.zeros_like(acc_ref)

    @pl.when(i < NB_BN)
    def _():
        u = u_ref[...]
        s = jnp.zeros((1, DH), jnp.float32)
        ss = jnp.zeros((1, DH), jnp.float32)
        for k in range(K):
            hk = u + vg_ref[:, k * DH:(k + 1) * DH]
            s = s + jnp.sum(hk, axis=0, keepdims=True)
            ss = ss + jnp.sum(hk * hk, axis=0, keepdims=True)
        acc_ref[0:1, 0:DH] = acc_ref[0:1, 0:DH] + s
        acc_ref[1:2, 0:DH] = acc_ref[1:2, 0:DH] + ss

    @pl.when(i >= NB_BN)
    def _():
        s = acc_ref[0:1, 0:DH]
        ss = acc_ref[1:2, 0:DH]
        m = s * (1.0 / NK)
        var = ss * (1.0 / NK) - m * m
        scale = g_ref[...] / jnp.sqrt(var + EPS)
        bias = be_ref[...] - m * scale
        u = u_ref[...]
        for k in range(K):
            hk = u + vg_ref[:, k * DH:(k + 1) * DH]
            hbuf_ref[k * R_BN:(k + 1) * R_BN, :] = jnp.maximum(
                hk * scale + bias, 0.0)
        z = lax.dot_general(hbuf_ref[...], w2_ref[...],
                            (((1,), (0,)), ((), ())),
                            preferred_element_type=jnp.float32) + b2_ref[...]
        r = z[0:R_BN, :]
        for k in range(1, K):
            r = jnp.maximum(r, z[k * R_BN:(k + 1) * R_BN, :])
        out_ref[...] = r
        outt_ref[...] = r.T


def _bn_mlp_max(vg, u, g1, be1, w2, b2):
    return pl.pallas_call(
        _bn_body,
        grid=(2 * NB_BN,),
        in_specs=[
            pl.BlockSpec((R_BN, K * DH), lambda i: (i % NB_BN, 0)),
            pl.BlockSpec((R_BN, DH), lambda i: (i % NB_BN, 0)),
            pl.BlockSpec((1, DH), lambda i: (0, 0)),
            pl.BlockSpec((1, DH), lambda i: (0, 0)),
            pl.BlockSpec((DH, DH), lambda i: (0, 0)),
            pl.BlockSpec((1, DH), lambda i: (0, 0)),
        ],
        out_specs=[
            pl.BlockSpec((R_BN, DH), lambda i: (i % NB_BN, 0)),
            pl.BlockSpec((DH, R_BN), lambda i: (0, i % NB_BN)),
        ],
        out_shape=[
            jax.ShapeDtypeStruct((N, DH), jnp.float32),
            jax.ShapeDtypeStruct((DH, N), jnp.float32),
        ],
        scratch_shapes=[
            pltpu.VMEM((8, 128), jnp.float32),
            pltpu.VMEM((K * R_BN, DH), jnp.float32),
        ],
    )(vg, u, g1, be1, w2, b2)


R_HD = 512
NB_HD = N // R_HD


def _head_body(x1_ref, x2_ref, x3_ref, w1_ref, b1_ref, w2_ref, b2_ref,
               w3_ref, b3_ref, w4_ref, b4_ref, out_ref):
    h = jnp.concatenate([x1_ref[...], x2_ref[...], x3_ref[...]], axis=1)
    h = jnp.maximum(lax.dot_general(h, w1_ref[...], (((1,), (0,)), ((), ())),
                                    preferred_element_type=jnp.float32)
                    + b1_ref[...], 0.0)
    h = jnp.maximum(lax.dot_general(h, w2_ref[...], (((1,), (0,)), ((), ())),
                                    preferred_element_type=jnp.float32)
                    + b2_ref[...], 0.0)
    h = jnp.maximum(lax.dot_general(h, w3_ref[...], (((1,), (0,)), ((), ())),
                                    preferred_element_type=jnp.float32)
                    + b3_ref[...], 0.0)
    h = lax.dot_general(h, w4_ref[...], (((1,), (0,)), ((), ())),
                        preferred_element_type=jnp.float32) + b4_ref[...]
    m = jnp.max(h, axis=1, keepdims=True)
    sh = h - m
    lse = jnp.log(jnp.sum(jnp.exp(sh), axis=1, keepdims=True))
    out_ref[...] = sh - lse


def _head(x1, x2, x3, w1, b1, w2, b2, w3, b3, w4, b4):
    dims = [192, 256, 128, 64, 16]
    full = lambda shape: pl.BlockSpec(shape, lambda i: (0, 0))
    return pl.pallas_call(
        _head_body,
        grid=(NB_HD,),
        in_specs=[
            pl.BlockSpec((R_HD, DH), lambda i: (i, 0)),
            pl.BlockSpec((R_HD, DH), lambda i: (i, 0)),
            pl.BlockSpec((R_HD, DH), lambda i: (i, 0)),
            full((dims[0], dims[1])), full((1, dims[1])),
            full((dims[1], dims[2])), full((1, dims[2])),
            full((dims[2], dims[3])), full((1, dims[3])),
            full((dims[3], dims[4])), full((1, dims[4])),
        ],
        out_specs=pl.BlockSpec((R_HD, dims[4]), lambda i: (i, 0)),
        out_shape=jax.ShapeDtypeStruct((N, dims[4]), jnp.float32),
    )(x1, x2, x3, w1, b1, w2, b2, w3, b3, w4, b4)


def _block_windows(batch):
    g_first = batch[R_KNN * jnp.arange(NB_KNN)]
    g_last = batch[R_KNN * jnp.arange(NB_KNN) + (R_KNN - 1)]
    starts = jnp.searchsorted(batch, g_first, side="left").astype(jnp.int32)
    ends = jnp.searchsorted(batch, g_last, side="right").astype(jnp.int32)
    lo = (starts // CW) * CW
    nch = (ends - lo + CW - 1) // CW
    return jnp.stack([lo, nch], axis=1)


def _edge_conv_layer(x, xt, batch_r, batch_c, desc, W1, b1, g1, be1, W2, b2):
    C = x.shape[1]
    idx, u, v = _knn_uv(x, xt, batch_r, batch_c, desc, W1, b1.reshape(1, DH), C)
    vg = _gather_rows(v, idx.reshape(-1))
    vg = vg.reshape(N, K * DH)
    return _bn_mlp_max(vg, u, g1.reshape(1, DH), be1.reshape(1, DH),
                       W2, b2.reshape(1, DH))


def kernel(x, batch, c1_W1, c1_b1, c1_g1, c1_be1, c1_W2, c1_b2,
           c2_W1, c2_b1, c2_g1, c2_be1, c2_W2, c2_b2,
           c3_W1, c3_b1, c3_g1, c3_be1, c3_W2, c3_b2,
           m_W1, m_b1, m_W2, m_b2, m_W3, m_b3, m_W4, m_b4):
    batch = batch.astype(jnp.int32)
    batch_r = batch.reshape(N, 1)
    batch_c = batch.reshape(1, N)
    desc = _block_windows(batch)
    x1, x1t = _edge_conv_layer(x, x.T, batch_r, batch_c, desc, c1_W1, c1_b1,
                               c1_g1, c1_be1, c1_W2, c1_b2)
    x2, x2t = _edge_conv_layer(x1, x1t, batch_r, batch_c, desc, c2_W1, c2_b1,
                               c2_g1, c2_be1, c2_W2, c2_b2)
    x3, _ = _edge_conv_layer(x2, x2t, batch_r, batch_c, desc, c3_W1, c3_b1,
                             c3_g1, c3_be1, c3_W2, c3_b2)
    return _head(x1, x2, x3, m_W1, m_b1.reshape(1, -1), m_W2,
                 m_b2.reshape(1, -1), m_W3, m_b3.reshape(1, -1),
                 m_W4, m_b4.reshape(1, -1))


# speedup vs baseline: 1.3707x; 1.3707x over previous
import functools

import jax
import jax.numpy as jnp
from jax import lax
from jax.experimental import pallas as pl
from jax.experimental.pallas import tpu as pltpu
from jax.experimental.pallas import tpu_sc as plsc

N = 4096
K = 16
DH = 64
EPS = 1e-5
NK = N * K
BIG = 1e30
BIG2 = 2e30

R_KNN = 512
NB_KNN = N // R_KNN


CW = 128
_WIDTHS = (1280, 2304, 4096)


def _select_topk(W, d_ref, idx_ref, lo):
    cols = lax.broadcasted_iota(jnp.int32, (R_KNN, W), 1)
    lanes = lax.broadcasted_iota(jnp.int32, (R_KNN, K), 1)

    def step(k, idxacc):
        dd = d_ref[:, :W]
        m = jnp.min(dd, axis=1, keepdims=True)
        cand = jnp.where(dd == m, cols, jnp.int32(N))
        a = jnp.min(cand, axis=1, keepdims=True)
        d_ref[:, :W] = jnp.where(cols == a, BIG2, dd)
        return jnp.where(lanes == k, a, idxacc)

    idxacc = lax.fori_loop(0, K, step, jnp.zeros((R_KNN, K), jnp.int32))
    idx_ref[...] = jnp.minimum(idxacc + lo, N - 1)


def _knn_uv_body(C, x_ref, xt_ref, br_ref, bc_ref, desc_ref, w1_ref, b1_ref,
                 idx_ref, u_ref, v_ref, d_ref):
    i = pl.program_id(0)
    lo = desc_ref[i, 0]
    nch = desc_ref[i, 1]
    xb = x_ref[...]
    br = br_ref[...]
    sq_r = jnp.sum(xb * xb, axis=1, keepdims=True)
    wsel = jnp.where(nch <= _WIDTHS[0] // CW, _WIDTHS[0],
                     jnp.where(nch <= _WIDTHS[1] // CW, _WIDTHS[1],
                               _WIDTHS[2]))

    for c in range(N // CW):
        @pl.when(c < nch)
        def _(c=c):
            off = pl.multiple_of(lo + c * CW, CW)
            xtc = xt_ref[:, pl.ds(off, CW)]
            p = lax.dot_general(xb, xtc, (((1,), (0,)), ((), ())),
                                preferred_element_type=jnp.float32)
            sq_c = jnp.sum(xtc * xtc, axis=0, keepdims=True)
            d = (sq_r + sq_c) - 2.0 * p
            mask = br != bc_ref[:, pl.ds(off, CW)]
            d_ref[:, c * CW:(c + 1) * CW] = jnp.where(mask, BIG, d)

        @pl.when((c >= nch) & (c * CW < wsel))
        def _(c=c):
            d_ref[:, c * CW:(c + 1) * CW] = jnp.full((R_KNN, CW), BIG,
                                                     jnp.float32)

    @pl.when(nch <= _WIDTHS[0] // CW)
    def _():
        _select_topk(_WIDTHS[0], d_ref, idx_ref, lo)

    @pl.when((nch > _WIDTHS[0] // CW) & (nch <= _WIDTHS[1] // CW))
    def _():
        _select_topk(_WIDTHS[1], d_ref, idx_ref, lo)

    @pl.when(nch > _WIDTHS[1] // CW)
    def _():
        _select_topk(_WIDTHS[2], d_ref, idx_ref, lo)

    w1 = w1_ref[...]
    wa = w1[0:C, :] - w1[C:2 * C, :]
    wb = w1[C:2 * C, :]
    u_ref[...] = lax.dot_general(xb, wa, (((1,), (0,)), ((), ())),
                                 preferred_element_type=jnp.float32) + b1_ref[...]
    v_ref[...] = lax.dot_general(xb, wb, (((1,), (0,)), ((), ())),
                                 preferred_element_type=jnp.float32)


def _knn_uv(x, xt, batch_r, batch_c, desc, w1, b1, C):
    return pl.pallas_call(
        functools.partial(_knn_uv_body, C),
        grid=(NB_KNN,),
        in_specs=[
            pl.BlockSpec((R_KNN, C), lambda i: (i, 0)),
            pl.BlockSpec((C, N), lambda i: (0, 0)),
            pl.BlockSpec((R_KNN, 1), lambda i: (i, 0)),
            pl.BlockSpec((1, N), lambda i: (0, 0)),
            pl.BlockSpec(memory_space=pltpu.SMEM),
            pl.BlockSpec((2 * C, DH), lambda i: (0, 0)),
            pl.BlockSpec((1, DH), lambda i: (0, 0)),
        ],
        out_specs=[
            pl.BlockSpec((R_KNN, K), lambda i: (i, 0)),
            pl.BlockSpec((R_KNN, DH), lambda i: (i, 0)),
            pl.BlockSpec((R_KNN, DH), lambda i: (i, 0)),
        ],
        out_shape=[
            jax.ShapeDtypeStruct((N, K), jnp.int32),
            jax.ShapeDtypeStruct((N, DH), jnp.float32),
            jax.ShapeDtypeStruct((N, DH), jnp.float32),
        ],
        scratch_shapes=[pltpu.VMEM((R_KNN, N), jnp.float32)],
    )(x, xt, batch_r, batch_c, desc, w1, b1)


SC_NC = 2
SC_NS = 16
SC_NW = SC_NC * SC_NS
GCH = 128
ROWS_PER_W = NK // SC_NW
N_CHUNKS = ROWS_PER_W // GCH


def _gather_body(v_hbm, idx_hbm, out_hbm, idx_v0, idx_v1, rows_v0, rows_v1,
                 gsem0, gsem1, ssem0, ssem1):
    wid = lax.axis_index("s") * SC_NC + lax.axis_index("c")
    idx_v = (idx_v0, idx_v1)
    rows_v = (rows_v0, rows_v1)
    gsem = (gsem0, gsem1)
    ssem = (ssem0, ssem1)
    base0 = wid * ROWS_PER_W
    pltpu.sync_copy(idx_hbm.at[pl.ds(base0, GCH)], idx_v[0])
    stores = [None, None]
    for c in range(N_CHUNKS):
        b = c & 1
        if stores[b] is not None:
            stores[b].wait()
        g = pltpu.async_copy(v_hbm.at[idx_v[b]], rows_v[b], gsem[b])
        if c + 1 < N_CHUNKS:
            pltpu.sync_copy(idx_hbm.at[pl.ds(base0 + (c + 1) * GCH, GCH)],
                            idx_v[1 - b])
        g.wait()
        st = pltpu.async_copy(rows_v[b], out_hbm.at[pl.ds(base0 + c * GCH, GCH)],
                              ssem[b])
        stores[b] = st
    for st in stores:
        st.wait()


@functools.lru_cache(maxsize=1)
def _build_gather():
    return functools.partial(
        pl.kernel,
        out_type=jax.ShapeDtypeStruct((NK, DH), jnp.float32),
        mesh=plsc.VectorSubcoreMesh(core_axis_name="c", subcore_axis_name="s"),
        scratch_types=[
            pltpu.VMEM((GCH,), jnp.int32),
            pltpu.VMEM((GCH,), jnp.int32),
            pltpu.VMEM((GCH, DH), jnp.float32),
            pltpu.VMEM((GCH, DH), jnp.float32),
            pltpu.SemaphoreType.DMA,
            pltpu.SemaphoreType.DMA,
            pltpu.SemaphoreType.DMA,
            pltpu.SemaphoreType.DMA,
        ],
        compiler_params=pltpu.CompilerParams(use_tc_tiling_on_sc=False),
    )(_gather_body)


def _gather_rows(v, idx_flat):
    return _build_gather()(v, idx_flat)


R_BN = 512
NB_BN = N // R_BN


def _bn_body(vg_ref, u_ref, g_ref, be_ref, w2_ref, b2_ref, out_ref, outt_ref,
             acc_ref, hbuf_ref):
    i = pl.program_id(0)

    @pl.when(i == 0)
    def _():
        acc_ref[...] = jnp.zeros_like(acc_ref)

    @pl.when(i < NB_BN)
    def _():
        u = u_ref[...]
        s = jnp.zeros((1, DH), jnp.float32)
        ss = jnp.zeros((1, DH), jnp.float32)
        for k in range(K):
            hk = u + vg_ref[:, k * DH:(k + 1) * DH]
            s = s + jnp.sum(hk, axis=0, keepdims=True)
            ss = ss + jnp.sum(hk * hk, axis=0, keepdims=True)
        acc_ref[0:1, 0:DH] = acc_ref[0:1, 0:DH] + s
        acc_ref[1:2, 0:DH] = acc_ref[1:2, 0:DH] + ss

    @pl.when(i >= NB_BN)
    def _():
        s = acc_ref[0:1, 0:DH]
        ss = acc_ref[1:2, 0:DH]
        m = s * (1.0 / NK)
        var = ss * (1.0 / NK) - m * m
        scale = g_ref[...] / jnp.sqrt(var + EPS)
        bias = be_ref[...] - m * scale
        u = u_ref[...]
        for k in range(K):
            hk = u + vg_ref[:, k * DH:(k + 1) * DH]
            hbuf_ref[k * R_BN:(k + 1) * R_BN, :] = jnp.maximum(
                hk * scale + bias, 0.0)
        z = lax.dot_general(hbuf_ref[...], w2_ref[...],
                            (((1,), (0,)), ((), ())),
                            preferred_element_type=jnp.float32) + b2_ref[...]
        r = z[0:R_BN, :]
        for k in range(1, K):
            r = jnp.maximum(r, z[k * R_BN:(k + 1) * R_BN, :])
        out_ref[...] = r
        outt_ref[...] = r.T


def _bn_mlp_max(vg, u, g1, be1, w2, b2):
    return pl.pallas_call(
        _bn_body,
        grid=(2 * NB_BN,),
        in_specs=[
            pl.BlockSpec((R_BN, K * DH), lambda i: (i % NB_BN, 0)),
            pl.BlockSpec((R_BN, DH), lambda i: (i % NB_BN, 0)),
            pl.BlockSpec((1, DH), lambda i: (0, 0)),
            pl.BlockSpec((1, DH), lambda i: (0, 0)),
            pl.BlockSpec((DH, DH), lambda i: (0, 0)),
            pl.BlockSpec((1, DH), lambda i: (0, 0)),
        ],
        out_specs=[
            pl.BlockSpec((R_BN, DH), lambda i: (i % NB_BN, 0)),
            pl.BlockSpec((DH, R_BN), lambda i: (0, i % NB_BN)),
        ],
        out_shape=[
            jax.ShapeDtypeStruct((N, DH), jnp.float32),
            jax.ShapeDtypeStruct((DH, N), jnp.float32),
        ],
        scratch_shapes=[
            pltpu.VMEM((8, 128), jnp.float32),
            pltpu.VMEM((K * R_BN, DH), jnp.float32),
        ],
    )(vg, u, g1, be1, w2, b2)


R_HD = 512
NB_HD = N // R_HD


def _head_body(x1_ref, x2_ref, x3_ref, w1_ref, b1_ref, w2_ref, b2_ref,
               w3_ref, b3_ref, w4_ref, b4_ref, out_ref):
    h = jnp.concatenate([x1_ref[...], x2_ref[...], x3_ref[...]], axis=1)
    h = jnp.maximum(lax.dot_general(h, w1_ref[...], (((1,), (0,)), ((), ())),
                                    preferred_element_type=jnp.float32)
                    + b1_ref[...], 0.0)
    h = jnp.maximum(lax.dot_general(h, w2_ref[...], (((1,), (0,)), ((), ())),
                                    preferred_element_type=jnp.float32)
                    + b2_ref[...], 0.0)
    h = jnp.maximum(lax.dot_general(h, w3_ref[...], (((1,), (0,)), ((), ())),
                                    preferred_element_type=jnp.float32)
                    + b3_ref[...], 0.0)
    h = lax.dot_general(h, w4_ref[...], (((1,), (0,)), ((), ())),
                        preferred_element_type=jnp.float32) + b4_ref[...]
    m = jnp.max(h, axis=1, keepdims=True)
    sh = h - m
    lse = jnp.log(jnp.sum(jnp.exp(sh), axis=1, keepdims=True))
    out_ref[...] = sh - lse


def _head(x1, x2, x3, w1, b1, w2, b2, w3, b3, w4, b4):
    dims = [192, 256, 128, 64, 16]
    full = lambda shape: pl.BlockSpec(shape, lambda i: (0, 0))
    return pl.pallas_call(
        _head_body,
        grid=(NB_HD,),
        in_specs=[
            pl.BlockSpec((R_HD, DH), lambda i: (i, 0)),
            pl.BlockSpec((R_HD, DH), lambda i: (i, 0)),
            pl.BlockSpec((R_HD, DH), lambda i: (i, 0)),
            full((dims[0], dims[1])), full((1, dims[1])),
            full((dims[1], dims[2])), full((1, dims[2])),
            full((dims[2], dims[3])), full((1, dims[3])),
            full((dims[3], dims[4])), full((1, dims[4])),
        ],
        out_specs=pl.BlockSpec((R_HD, dims[4]), lambda i: (i, 0)),
        out_shape=jax.ShapeDtypeStruct((N, dims[4]), jnp.float32),
    )(x1, x2, x3, w1, b1, w2, b2, w3, b3, w4, b4)


def _block_windows(batch):
    g_first = batch[R_KNN * jnp.arange(NB_KNN)]
    g_last = batch[R_KNN * jnp.arange(NB_KNN) + (R_KNN - 1)]
    starts = jnp.searchsorted(batch, g_first, side="left").astype(jnp.int32)
    ends = jnp.searchsorted(batch, g_last, side="right").astype(jnp.int32)
    lo = (starts // CW) * CW
    nch = (ends - lo + CW - 1) // CW
    return jnp.stack([lo, nch], axis=1)


def _edge_conv_layer(x, xt, batch_r, batch_c, desc, W1, b1, g1, be1, W2, b2):
    C = x.shape[1]
    idx, u, v = _knn_uv(x, xt, batch_r, batch_c, desc, W1, b1.reshape(1, DH), C)
    vg = _gather_rows(v, idx.reshape(-1))
    vg = vg.reshape(N, K * DH)
    return _bn_mlp_max(vg, u, g1.reshape(1, DH), be1.reshape(1, DH),
                       W2, b2.reshape(1, DH))


def kernel(x, batch, c1_W1, c1_b1, c1_g1, c1_be1, c1_W2, c1_b2,
           c2_W1, c2_b1, c2_g1, c2_be1, c2_W2, c2_b2,
           c3_W1, c3_b1, c3_g1, c3_be1, c3_W2, c3_b2,
           m_W1, m_b1, m_W2, m_b2, m_W3, m_b3, m_W4, m_b4):
    batch = batch.astype(jnp.int32)
    batch_r = batch.reshape(N, 1)
    batch_c = batch.reshape(1, N)
    desc = _block_windows(batch)
    x1, x1t = _edge_conv_layer(x, x.T, batch_r, batch_c, desc, c1_W1, c1_b1,
                               c1_g1, c1_be1, c1_W2, c1_b2)
    x2, x2t = _edge_conv_layer(x1, x1t, batch_r, batch_c, desc, c2_W1, c2_b1,
                               c2_g1, c2_be1, c2_W2, c2_b2)
    x3, _ = _edge_conv_layer(x2, x2t, batch_r, batch_c, desc, c3_W1, c3_b1,
                             c3_g1, c3_be1, c3_W2, c3_b2)
    return _head(x1, x2, x3, m_W1, m_b1.reshape(1, -1), m_W2,
                 m_b2.reshape(1, -1), m_W3, m_b3.reshape(1, -1),
                 m_W4, m_b4.reshape(1, -1))


# speedup vs baseline: 1.4562x; 1.0624x over previous
import functools

import jax
import jax.numpy as jnp
from jax import lax
from jax.experimental import pallas as pl
from jax.experimental.pallas import tpu as pltpu
from jax.experimental.pallas import tpu_sc as plsc

N = 4096
K = 16
DH = 64
EPS = 1e-5
NK = N * K
BIG = 1e30
BIG2 = 2e30

R_KNN = 512
NB_KNN = N // R_KNN


CW = 128
_WIDTHS = (1280, 2304, 4096)


def _select_topk(W, d_ref, idx_ref, lo):
    cols = lax.broadcasted_iota(jnp.int32, (R_KNN, W), 1)
    lanes = lax.broadcasted_iota(jnp.int32, (R_KNN, K), 1)

    def step(k, idxacc):
        dd = d_ref[:, :W]
        m = jnp.min(dd, axis=1, keepdims=True)
        cand = jnp.where(dd == m, cols, jnp.int32(N))
        a = jnp.min(cand, axis=1, keepdims=True)
        d_ref[:, :W] = jnp.where(cols == a, BIG2, dd)
        return jnp.where(lanes == k, a, idxacc)

    idxacc = lax.fori_loop(0, K, step, jnp.zeros((R_KNN, K), jnp.int32))
    idx_ref[...] = jnp.minimum(idxacc + lo, N - 1)


def _knn_uv_body(C, x_ref, xt_ref, br_ref, bc_ref, desc_ref, w1_ref, b1_ref,
                 idx_ref, u_ref, v_ref, d_ref):
    i = pl.program_id(0)
    lo = desc_ref[i, 0]
    nch = desc_ref[i, 1]
    xb = x_ref[...]
    br = br_ref[...]
    sq_r = jnp.sum(xb * xb, axis=1, keepdims=True)
    def fill_and_select(W):
        ls = pl.multiple_of(jnp.minimum(lo, N - W), CW)
        shift = lo - ls
        xtw = xt_ref[:, pl.ds(ls, W)]
        p = lax.dot_general(xb, xtw, (((1,), (0,)), ((), ())),
                            preferred_element_type=jnp.float32)
        sq_c = jnp.sum(xtw * xtw, axis=0, keepdims=True)
        d = (sq_r + sq_c) - 2.0 * p
        cols = lax.broadcasted_iota(jnp.int32, (R_KNN, W), 1)
        valid = ((cols >= shift) & (cols < shift + nch * CW)
                 & (br == bc_ref[:, pl.ds(ls, W)]))
        d_ref[:, :W] = jnp.where(valid, d, BIG)
        _select_topk(W, d_ref, idx_ref, ls)

    @pl.when(nch <= _WIDTHS[0] // CW)
    def _():
        fill_and_select(_WIDTHS[0])

    @pl.when((nch > _WIDTHS[0] // CW) & (nch <= _WIDTHS[1] // CW))
    def _():
        fill_and_select(_WIDTHS[1])

    @pl.when(nch > _WIDTHS[1] // CW)
    def _():
        fill_and_select(_WIDTHS[2])

    w1 = w1_ref[...]
    wa = w1[0:C, :] - w1[C:2 * C, :]
    wb = w1[C:2 * C, :]
    u_ref[...] = lax.dot_general(xb, wa, (((1,), (0,)), ((), ())),
                                 preferred_element_type=jnp.float32) + b1_ref[...]
    v_ref[...] = lax.dot_general(xb, wb, (((1,), (0,)), ((), ())),
                                 preferred_element_type=jnp.float32)


def _knn_uv(x, xt, batch_r, batch_c, desc, w1, b1, C):
    return pl.pallas_call(
        functools.partial(_knn_uv_body, C),
        grid=(NB_KNN,),
        in_specs=[
            pl.BlockSpec((R_KNN, C), lambda i: (i, 0)),
            pl.BlockSpec((C, N), lambda i: (0, 0)),
            pl.BlockSpec((R_KNN, 1), lambda i: (i, 0)),
            pl.BlockSpec((1, N), lambda i: (0, 0)),
            pl.BlockSpec(memory_space=pltpu.SMEM),
            pl.BlockSpec((2 * C, DH), lambda i: (0, 0)),
            pl.BlockSpec((1, DH), lambda i: (0, 0)),
        ],
        out_specs=[
            pl.BlockSpec((R_KNN, K), lambda i: (i, 0)),
            pl.BlockSpec((R_KNN, DH), lambda i: (i, 0)),
            pl.BlockSpec((R_KNN, DH), lambda i: (i, 0)),
        ],
        out_shape=[
            jax.ShapeDtypeStruct((N, K), jnp.int32),
            jax.ShapeDtypeStruct((N, DH), jnp.float32),
            jax.ShapeDtypeStruct((N, DH), jnp.float32),
        ],
        scratch_shapes=[pltpu.VMEM((R_KNN, N), jnp.float32)],
    )(x, xt, batch_r, batch_c, desc, w1, b1)


SC_NC = 2
SC_NS = 16
SC_NW = SC_NC * SC_NS
GCH = 128
ROWS_PER_W = NK // SC_NW
N_CHUNKS = ROWS_PER_W // GCH


def _gather_body(v_hbm, idx_hbm, out_hbm, idx_v0, idx_v1, rows_v0, rows_v1,
                 gsem0, gsem1, ssem0, ssem1):
    wid = lax.axis_index("s") * SC_NC + lax.axis_index("c")
    idx_v = (idx_v0, idx_v1)
    rows_v = (rows_v0, rows_v1)
    gsem = (gsem0, gsem1)
    ssem = (ssem0, ssem1)
    base0 = wid * ROWS_PER_W
    pltpu.sync_copy(idx_hbm.at[pl.ds(base0, GCH)], idx_v[0])
    stores = [None, None]
    for c in range(N_CHUNKS):
        b = c & 1
        if stores[b] is not None:
            stores[b].wait()
        g = pltpu.async_copy(v_hbm.at[idx_v[b]], rows_v[b], gsem[b])
        if c + 1 < N_CHUNKS:
            pltpu.sync_copy(idx_hbm.at[pl.ds(base0 + (c + 1) * GCH, GCH)],
                            idx_v[1 - b])
        g.wait()
        st = pltpu.async_copy(rows_v[b], out_hbm.at[pl.ds(base0 + c * GCH, GCH)],
                              ssem[b])
        stores[b] = st
    for st in stores:
        st.wait()


@functools.lru_cache(maxsize=1)
def _build_gather():
    return functools.partial(
        pl.kernel,
        out_type=jax.ShapeDtypeStruct((NK, DH), jnp.float32),
        mesh=plsc.VectorSubcoreMesh(core_axis_name="c", subcore_axis_name="s"),
        scratch_types=[
            pltpu.VMEM((GCH,), jnp.int32),
            pltpu.VMEM((GCH,), jnp.int32),
            pltpu.VMEM((GCH, DH), jnp.float32),
            pltpu.VMEM((GCH, DH), jnp.float32),
            pltpu.SemaphoreType.DMA,
            pltpu.SemaphoreType.DMA,
            pltpu.SemaphoreType.DMA,
            pltpu.SemaphoreType.DMA,
        ],
        compiler_params=pltpu.CompilerParams(use_tc_tiling_on_sc=False),
    )(_gather_body)


def _gather_rows(v, idx_flat):
    return _build_gather()(v, idx_flat)


R_BN = 512
NB_BN = N // R_BN


def _bn_body(vg_ref, u_ref, g_ref, be_ref, w2_ref, b2_ref, out_ref, outt_ref,
             acc_ref, hbuf_ref):
    i = pl.program_id(0)

    @pl.when(i == 0)
    def _():
        acc_ref[...] = jnp.zeros_like(acc_ref)

    @pl.when(i < NB_BN)
    def _():
        u = u_ref[...]
        s = jnp.zeros((1, DH), jnp.float32)
        ss = jnp.zeros((1, DH), jnp.float32)
        for k in range(K):
            hk = u + vg_ref[:, k * DH:(k + 1) * DH]
            s = s + jnp.sum(hk, axis=0, keepdims=True)
            ss = ss + jnp.sum(hk * hk, axis=0, keepdims=True)
        acc_ref[0:1, 0:DH] = acc_ref[0:1, 0:DH] + s
        acc_ref[1:2, 0:DH] = acc_ref[1:2, 0:DH] + ss

    @pl.when(i >= NB_BN)
    def _():
        s = acc_ref[0:1, 0:DH]
        ss = acc_ref[1:2, 0:DH]
        m = s * (1.0 / NK)
        var = ss * (1.0 / NK) - m * m
        scale = g_ref[...] / jnp.sqrt(var + EPS)
        bias = be_ref[...] - m * scale
        u = u_ref[...]
        for k in range(K):
            hk = u + vg_ref[:, k * DH:(k + 1) * DH]
            hbuf_ref[k * R_BN:(k + 1) * R_BN, :] = jnp.maximum(
                hk * scale + bias, 0.0)
        z = lax.dot_general(hbuf_ref[...], w2_ref[...],
                            (((1,), (0,)), ((), ())),
                            preferred_element_type=jnp.float32) + b2_ref[...]
        r = z[0:R_BN, :]
        for k in range(1, K):
            r = jnp.maximum(r, z[k * R_BN:(k + 1) * R_BN, :])
        out_ref[...] = r
        outt_ref[...] = r.T


def _bn_mlp_max(vg, u, g1, be1, w2, b2):
    return pl.pallas_call(
        _bn_body,
        grid=(2 * NB_BN,),
        in_specs=[
            pl.BlockSpec((R_BN, K * DH), lambda i: (i % NB_BN, 0)),
            pl.BlockSpec((R_BN, DH), lambda i: (i % NB_BN, 0)),
            pl.BlockSpec((1, DH), lambda i: (0, 0)),
            pl.BlockSpec((1, DH), lambda i: (0, 0)),
            pl.BlockSpec((DH, DH), lambda i: (0, 0)),
            pl.BlockSpec((1, DH), lambda i: (0, 0)),
        ],
        out_specs=[
            pl.BlockSpec((R_BN, DH), lambda i: (i % NB_BN, 0)),
            pl.BlockSpec((DH, R_BN), lambda i: (0, i % NB_BN)),
        ],
        out_shape=[
            jax.ShapeDtypeStruct((N, DH), jnp.float32),
            jax.ShapeDtypeStruct((DH, N), jnp.float32),
        ],
        scratch_shapes=[
            pltpu.VMEM((8, 128), jnp.float32),
            pltpu.VMEM((K * R_BN, DH), jnp.float32),
        ],
    )(vg, u, g1, be1, w2, b2)


R_HD = 512
NB_HD = N // R_HD


def _head_body(x1_ref, x2_ref, x3_ref, w1_ref, b1_ref, w2_ref, b2_ref,
               w3_ref, b3_ref, w4_ref, b4_ref, out_ref):
    h = jnp.concatenate([x1_ref[...], x2_ref[...], x3_ref[...]], axis=1)
    h = jnp.maximum(lax.dot_general(h, w1_ref[...], (((1,), (0,)), ((), ())),
                                    preferred_element_type=jnp.float32)
                    + b1_ref[...], 0.0)
    h = jnp.maximum(lax.dot_general(h, w2_ref[...], (((1,), (0,)), ((), ())),
                                    preferred_element_type=jnp.float32)
                    + b2_ref[...], 0.0)
    h = jnp.maximum(lax.dot_general(h, w3_ref[...], (((1,), (0,)), ((), ())),
                                    preferred_element_type=jnp.float32)
                    + b3_ref[...], 0.0)
    h = lax.dot_general(h, w4_ref[...], (((1,), (0,)), ((), ())),
                        preferred_element_type=jnp.float32) + b4_ref[...]
    m = jnp.max(h, axis=1, keepdims=True)
    sh = h - m
    lse = jnp.log(jnp.sum(jnp.exp(sh), axis=1, keepdims=True))
    out_ref[...] = sh - lse


def _head(x1, x2, x3, w1, b1, w2, b2, w3, b3, w4, b4):
    dims = [192, 256, 128, 64, 16]
    full = lambda shape: pl.BlockSpec(shape, lambda i: (0, 0))
    return pl.pallas_call(
        _head_body,
        grid=(NB_HD,),
        in_specs=[
            pl.BlockSpec((R_HD, DH), lambda i: (i, 0)),
            pl.BlockSpec((R_HD, DH), lambda i: (i, 0)),
            pl.BlockSpec((R_HD, DH), lambda i: (i, 0)),
            full((dims[0], dims[1])), full((1, dims[1])),
            full((dims[1], dims[2])), full((1, dims[2])),
            full((dims[2], dims[3])), full((1, dims[3])),
            full((dims[3], dims[4])), full((1, dims[4])),
        ],
        out_specs=pl.BlockSpec((R_HD, dims[4]), lambda i: (i, 0)),
        out_shape=jax.ShapeDtypeStruct((N, dims[4]), jnp.float32),
    )(x1, x2, x3, w1, b1, w2, b2, w3, b3, w4, b4)


def _block_windows(batch):
    g_first = batch[R_KNN * jnp.arange(NB_KNN)]
    g_last = batch[R_KNN * jnp.arange(NB_KNN) + (R_KNN - 1)]
    starts = jnp.searchsorted(batch, g_first, side="left").astype(jnp.int32)
    ends = jnp.searchsorted(batch, g_last, side="right").astype(jnp.int32)
    lo = (starts // CW) * CW
    nch = (ends - lo + CW - 1) // CW
    return jnp.stack([lo, nch], axis=1)


def _edge_conv_layer(x, xt, batch_r, batch_c, desc, W1, b1, g1, be1, W2, b2):
    C = x.shape[1]
    idx, u, v = _knn_uv(x, xt, batch_r, batch_c, desc, W1, b1.reshape(1, DH), C)
    vg = _gather_rows(v, idx.reshape(-1))
    vg = vg.reshape(N, K * DH)
    return _bn_mlp_max(vg, u, g1.reshape(1, DH), be1.reshape(1, DH),
                       W2, b2.reshape(1, DH))


def kernel(x, batch, c1_W1, c1_b1, c1_g1, c1_be1, c1_W2, c1_b2,
           c2_W1, c2_b1, c2_g1, c2_be1, c2_W2, c2_b2,
           c3_W1, c3_b1, c3_g1, c3_be1, c3_W2, c3_b2,
           m_W1, m_b1, m_W2, m_b2, m_W3, m_b3, m_W4, m_b4):
    batch = batch.astype(jnp.int32)
    batch_r = batch.reshape(N, 1)
    batch_c = batch.reshape(1, N)
    desc = _block_windows(batch)
    x1, x1t = _edge_conv_layer(x, x.T, batch_r, batch_c, desc, c1_W1, c1_b1,
                               c1_g1, c1_be1, c1_W2, c1_b2)
    x2, x2t = _edge_conv_layer(x1, x1t, batch_r, batch_c, desc, c2_W1, c2_b1,
                               c2_g1, c2_be1, c2_W2, c2_b2)
    x3, _ = _edge_conv_layer(x2, x2t, batch_r, batch_c, desc, c3_W1, c3_b1,
                             c3_g1, c3_be1, c3_W2, c3_b2)
    return _head(x1, x2, x3, m_W1, m_b1.reshape(1, -1), m_W2,
                 m_b2.reshape(1, -1), m_W3, m_b3.reshape(1, -1),
                 m_W4, m_b4.reshape(1, -1))


# speedup vs baseline: 1.4929x; 1.0252x over previous
import functools

import jax
import jax.numpy as jnp
from jax import lax
from jax.experimental import pallas as pl
from jax.experimental.pallas import tpu as pltpu
from jax.experimental.pallas import tpu_sc as plsc

N = 4096
K = 16
DH = 64
EPS = 1e-5
NK = N * K
BIG = 1e30
BIG2 = 2e30

R_KNN = 512
NB_KNN = N // R_KNN


CW = 128
_WIDTHS = (1280, 2304, 4096)


def _select_topk(W, d_ref, idx_ref, lo):
    cols = lax.broadcasted_iota(jnp.int32, (R_KNN, W), 1)
    lanes = lax.broadcasted_iota(jnp.int32, (R_KNN, K), 1)

    def step(k, idxacc):
        dd = d_ref[:, :W]
        m = jnp.min(dd, axis=1, keepdims=True)
        cand = jnp.where(dd == m, cols, jnp.int32(N))
        a = jnp.min(cand, axis=1, keepdims=True)
        d_ref[:, :W] = jnp.where(cols == a, BIG2, dd)
        return jnp.where(lanes == k, a, idxacc)

    idxacc = lax.fori_loop(0, K, step, jnp.zeros((R_KNN, K), jnp.int32))
    idx_ref[...] = jnp.minimum(idxacc + lo, N - 1)


def _knn_uv_body(C, x_ref, xt_ref, br_ref, bc_ref, desc_ref, w1_ref, b1_ref,
                 idx_ref, u_ref, v_ref, d_ref):
    i = pl.program_id(0)
    lo = desc_ref[i, 0]
    nch = desc_ref[i, 1]
    xb = x_ref[...]
    br = br_ref[...]
    sq_r = jnp.sum(xb * xb, axis=1, keepdims=True)
    def fill_and_select(W):
        ls = pl.multiple_of(jnp.minimum(lo, N - W), CW)
        shift = lo - ls
        xtw = xt_ref[:, pl.ds(ls, W)]
        p = lax.dot_general(xb, xtw, (((1,), (0,)), ((), ())),
                            preferred_element_type=jnp.float32)
        sq_c = jnp.sum(xtw * xtw, axis=0, keepdims=True)
        d = (sq_r + sq_c) - 2.0 * p
        cols = lax.broadcasted_iota(jnp.int32, (R_KNN, W), 1)
        valid = ((cols >= shift) & (cols < shift + nch * CW)
                 & (br == bc_ref[:, pl.ds(ls, W)]))
        d_ref[:, :W] = jnp.where(valid, d, BIG)
        _select_topk(W, d_ref, idx_ref, ls)

    @pl.when(nch <= _WIDTHS[0] // CW)
    def _():
        fill_and_select(_WIDTHS[0])

    @pl.when((nch > _WIDTHS[0] // CW) & (nch <= _WIDTHS[1] // CW))
    def _():
        fill_and_select(_WIDTHS[1])

    @pl.when(nch > _WIDTHS[1] // CW)
    def _():
        fill_and_select(_WIDTHS[2])

    w1 = w1_ref[...]
    wa = w1[0:C, :] - w1[C:2 * C, :]
    wb = w1[C:2 * C, :]
    u_ref[...] = lax.dot_general(xb, wa, (((1,), (0,)), ((), ())),
                                 preferred_element_type=jnp.float32) + b1_ref[...]
    v_ref[...] = lax.dot_general(xb, wb, (((1,), (0,)), ((), ())),
                                 preferred_element_type=jnp.float32)


def _knn_uv(x, xt, batch_r, batch_c, desc, w1, b1, C):
    return pl.pallas_call(
        functools.partial(_knn_uv_body, C),
        grid=(NB_KNN,),
        in_specs=[
            pl.BlockSpec((R_KNN, C), lambda i: (i, 0)),
            pl.BlockSpec((C, N), lambda i: (0, 0)),
            pl.BlockSpec((R_KNN, 1), lambda i: (i, 0)),
            pl.BlockSpec((1, N), lambda i: (0, 0)),
            pl.BlockSpec(memory_space=pltpu.SMEM),
            pl.BlockSpec((2 * C, DH), lambda i: (0, 0)),
            pl.BlockSpec((1, DH), lambda i: (0, 0)),
        ],
        out_specs=[
            pl.BlockSpec((R_KNN, K), lambda i: (i, 0)),
            pl.BlockSpec((R_KNN, DH), lambda i: (i, 0)),
            pl.BlockSpec((R_KNN, DH), lambda i: (i, 0)),
        ],
        out_shape=[
            jax.ShapeDtypeStruct((N, K), jnp.int32),
            jax.ShapeDtypeStruct((N, DH), jnp.float32),
            jax.ShapeDtypeStruct((N, DH), jnp.float32),
        ],
        scratch_shapes=[pltpu.VMEM((R_KNN, N), jnp.float32)],
    )(x, xt, batch_r, batch_c, desc, w1, b1)


SC_NC = 2
SC_NS = 16
SC_NW = SC_NC * SC_NS
GCH = 128
ROWS_PER_W = NK // SC_NW
N_CHUNKS = ROWS_PER_W // GCH


def _gather_body(v_hbm, idx_hbm, out_hbm, idx_v, rows_v0, rows_v1, rows_v2,
                 rows_v3, gsem0, gsem1, gsem2, gsem3, ssem0, ssem1, ssem2,
                 ssem3):
    wid = lax.axis_index("s") * SC_NC + lax.axis_index("c")
    rows_v = (rows_v0, rows_v1, rows_v2, rows_v3)
    gsem = (gsem0, gsem1, gsem2, gsem3)
    ssem = (ssem0, ssem1, ssem2, ssem3)
    base0 = wid * ROWS_PER_W
    pltpu.sync_copy(idx_hbm.at[pl.ds(base0, ROWS_PER_W)], idx_v)

    def start_gather(c):
        b = c % 4
        return pltpu.async_copy(
            v_hbm.at[idx_v.at[pl.ds(c * GCH, GCH)]], rows_v[b], gsem[b])

    gathers = [start_gather(0), start_gather(1)]
    stores = [None, None, None, None]
    for c in range(N_CHUNKS):
        b = c % 4
        if c + 2 < N_CHUNKS:
            b2 = (c + 2) % 4
            if stores[b2] is not None:
                stores[b2].wait()
            gathers.append(start_gather(c + 2))
        gathers[c].wait()
        stores[b] = pltpu.async_copy(
            rows_v[b], out_hbm.at[pl.ds(base0 + c * GCH, GCH)], ssem[b])
    for st in stores:
        st.wait()


@functools.lru_cache(maxsize=1)
def _build_gather():
    return functools.partial(
        pl.kernel,
        out_type=jax.ShapeDtypeStruct((NK, DH), jnp.float32),
        mesh=plsc.VectorSubcoreMesh(core_axis_name="c", subcore_axis_name="s"),
        scratch_types=[
            pltpu.VMEM((ROWS_PER_W,), jnp.int32),
            pltpu.VMEM((GCH, DH), jnp.float32),
            pltpu.VMEM((GCH, DH), jnp.float32),
            pltpu.VMEM((GCH, DH), jnp.float32),
            pltpu.VMEM((GCH, DH), jnp.float32),
            pltpu.SemaphoreType.DMA,
            pltpu.SemaphoreType.DMA,
            pltpu.SemaphoreType.DMA,
            pltpu.SemaphoreType.DMA,
            pltpu.SemaphoreType.DMA,
            pltpu.SemaphoreType.DMA,
            pltpu.SemaphoreType.DMA,
            pltpu.SemaphoreType.DMA,
        ],
        compiler_params=pltpu.CompilerParams(use_tc_tiling_on_sc=False),
    )(_gather_body)


def _gather_rows(v, idx_flat):
    return _build_gather()(v, idx_flat)


R_BN = 512
NB_BN = N // R_BN


def _bn_body(vg_ref, u_ref, g_ref, be_ref, w2_ref, b2_ref, out_ref, outt_ref,
             acc_ref, hbuf_ref):
    i = pl.program_id(0)

    @pl.when(i == 0)
    def _():
        acc_ref[...] = jnp.zeros_like(acc_ref)

    @pl.when(i < NB_BN)
    def _():
        u = u_ref[...]
        s = jnp.zeros((1, DH), jnp.float32)
        ss = jnp.zeros((1, DH), jnp.float32)
        for k in range(K):
            hk = u + vg_ref[:, k * DH:(k + 1) * DH]
            s = s + jnp.sum(hk, axis=0, keepdims=True)
            ss = ss + jnp.sum(hk * hk, axis=0, keepdims=True)
        acc_ref[0:1, 0:DH] = acc_ref[0:1, 0:DH] + s
        acc_ref[1:2, 0:DH] = acc_ref[1:2, 0:DH] + ss

    @pl.when(i >= NB_BN)
    def _():
        s = acc_ref[0:1, 0:DH]
        ss = acc_ref[1:2, 0:DH]
        m = s * (1.0 / NK)
        var = ss * (1.0 / NK) - m * m
        scale = g_ref[...] / jnp.sqrt(var + EPS)
        bias = be_ref[...] - m * scale
        u = u_ref[...]
        for k in range(K):
            hk = u + vg_ref[:, k * DH:(k + 1) * DH]
            hbuf_ref[k * R_BN:(k + 1) * R_BN, :] = jnp.maximum(
                hk * scale + bias, 0.0)
        z = lax.dot_general(hbuf_ref[...], w2_ref[...],
                            (((1,), (0,)), ((), ())),
                            preferred_element_type=jnp.float32) + b2_ref[...]
        r = z[0:R_BN, :]
        for k in range(1, K):
            r = jnp.maximum(r, z[k * R_BN:(k + 1) * R_BN, :])
        out_ref[...] = r
        outt_ref[...] = r.T


def _bn_mlp_max(vg, u, g1, be1, w2, b2):
    return pl.pallas_call(
        _bn_body,
        grid=(2 * NB_BN,),
        in_specs=[
            pl.BlockSpec((R_BN, K * DH), lambda i: (i % NB_BN, 0)),
            pl.BlockSpec((R_BN, DH), lambda i: (i % NB_BN, 0)),
            pl.BlockSpec((1, DH), lambda i: (0, 0)),
            pl.BlockSpec((1, DH), lambda i: (0, 0)),
            pl.BlockSpec((DH, DH), lambda i: (0, 0)),
            pl.BlockSpec((1, DH), lambda i: (0, 0)),
        ],
        out_specs=[
            pl.BlockSpec((R_BN, DH), lambda i: (i % NB_BN, 0)),
            pl.BlockSpec((DH, R_BN), lambda i: (0, i % NB_BN)),
        ],
        out_shape=[
            jax.ShapeDtypeStruct((N, DH), jnp.float32),
            jax.ShapeDtypeStruct((DH, N), jnp.float32),
        ],
        scratch_shapes=[
            pltpu.VMEM((8, 128), jnp.float32),
            pltpu.VMEM((K * R_BN, DH), jnp.float32),
        ],
    )(vg, u, g1, be1, w2, b2)


R_HD = 512
NB_HD = N // R_HD


def _head_body(x1_ref, x2_ref, x3_ref, w1_ref, b1_ref, w2_ref, b2_ref,
               w3_ref, b3_ref, w4_ref, b4_ref, out_ref):
    h = jnp.concatenate([x1_ref[...], x2_ref[...], x3_ref[...]], axis=1)
    h = jnp.maximum(lax.dot_general(h, w1_ref[...], (((1,), (0,)), ((), ())),
                                    preferred_element_type=jnp.float32)
                    + b1_ref[...], 0.0)
    h = jnp.maximum(lax.dot_general(h, w2_ref[...], (((1,), (0,)), ((), ())),
                                    preferred_element_type=jnp.float32)
                    + b2_ref[...], 0.0)
    h = jnp.maximum(lax.dot_general(h, w3_ref[...], (((1,), (0,)), ((), ())),
                                    preferred_element_type=jnp.float32)
                    + b3_ref[...], 0.0)
    h = lax.dot_general(h, w4_ref[...], (((1,), (0,)), ((), ())),
                        preferred_element_type=jnp.float32) + b4_ref[...]
    m = jnp.max(h, axis=1, keepdims=True)
    sh = h - m
    lse = jnp.log(jnp.sum(jnp.exp(sh), axis=1, keepdims=True))
    out_ref[...] = sh - lse


def _head(x1, x2, x3, w1, b1, w2, b2, w3, b3, w4, b4):
    dims = [192, 256, 128, 64, 16]
    full = lambda shape: pl.BlockSpec(shape, lambda i: (0, 0))
    return pl.pallas_call(
        _head_body,
        grid=(NB_HD,),
        in_specs=[
            pl.BlockSpec((R_HD, DH), lambda i: (i, 0)),
            pl.BlockSpec((R_HD, DH), lambda i: (i, 0)),
            pl.BlockSpec((R_HD, DH), lambda i: (i, 0)),
            full((dims[0], dims[1])), full((1, dims[1])),
            full((dims[1], dims[2])), full((1, dims[2])),
            full((dims[2], dims[3])), full((1, dims[3])),
            full((dims[3], dims[4])), full((1, dims[4])),
        ],
        out_specs=pl.BlockSpec((R_HD, dims[4]), lambda i: (i, 0)),
        out_shape=jax.ShapeDtypeStruct((N, dims[4]), jnp.float32),
    )(x1, x2, x3, w1, b1, w2, b2, w3, b3, w4, b4)


def _block_windows(batch):
    g_first = batch[R_KNN * jnp.arange(NB_KNN)]
    g_last = batch[R_KNN * jnp.arange(NB_KNN) + (R_KNN - 1)]
    starts = jnp.searchsorted(batch, g_first, side="left").astype(jnp.int32)
    ends = jnp.searchsorted(batch, g_last, side="right").astype(jnp.int32)
    lo = (starts // CW) * CW
    nch = (ends - lo + CW - 1) // CW
    return jnp.stack([lo, nch], axis=1)


def _edge_conv_layer(x, xt, batch_r, batch_c, desc, W1, b1, g1, be1, W2, b2):
    C = x.shape[1]
    idx, u, v = _knn_uv(x, xt, batch_r, batch_c, desc, W1, b1.reshape(1, DH), C)
    vg = _gather_rows(v, idx.reshape(-1))
    vg = vg.reshape(N, K * DH)
    return _bn_mlp_max(vg, u, g1.reshape(1, DH), be1.reshape(1, DH),
                       W2, b2.reshape(1, DH))


def kernel(x, batch, c1_W1, c1_b1, c1_g1, c1_be1, c1_W2, c1_b2,
           c2_W1, c2_b1, c2_g1, c2_be1, c2_W2, c2_b2,
           c3_W1, c3_b1, c3_g1, c3_be1, c3_W2, c3_b2,
           m_W1, m_b1, m_W2, m_b2, m_W3, m_b3, m_W4, m_b4):
    batch = batch.astype(jnp.int32)
    batch_r = batch.reshape(N, 1)
    batch_c = batch.reshape(1, N)
    desc = _block_windows(batch)
    x1, x1t = _edge_conv_layer(x, x.T, batch_r, batch_c, desc, c1_W1, c1_b1,
                               c1_g1, c1_be1, c1_W2, c1_b2)
    x2, x2t = _edge_conv_layer(x1, x1t, batch_r, batch_c, desc, c2_W1, c2_b1,
                               c2_g1, c2_be1, c2_W2, c2_b2)
    x3, _ = _edge_conv_layer(x2, x2t, batch_r, batch_c, desc, c3_W1, c3_b1,
                             c3_g1, c3_be1, c3_W2, c3_b2)
    return _head(x1, x2, x3, m_W1, m_b1.reshape(1, -1), m_W2,
                 m_b2.reshape(1, -1), m_W3, m_b3.reshape(1, -1),
                 m_W4, m_b4.reshape(1, -1))
